# baseline (speedup 1.0000x reference)
import functools

import jax
import jax.numpy as jnp
from jax.experimental import pallas as pl
from jax.experimental.pallas import tpu as pltpu

_UNROLL = 8


def _gather_sum_kernel(ids_ref, tbl_ref, o_ref, slab_ref, *, tile_s):
    s = pl.program_id(0)
    l = pl.program_id(1)
    base = s * tile_s

    def chunk(jo, carry):
        j0 = jo * _UNROLL
        rows = [tbl_ref[ids_ref[l, base + j0 + u], 0] for u in range(_UNROLL)]
        for u in range(_UNROLL):
            slab_ref[j0 + u, 0] = rows[u]
        return carry

    jax.lax.fori_loop(0, tile_s // _UNROLL, chunk, 0)

    @pl.when(l == 0)
    def _():
        o_ref[...] = slab_ref[...]

    @pl.when(l > 0)
    def _():
        o_ref[...] += slab_ref[...]


@functools.partial(jax.jit, static_argnames=("vocab",))
def _embed_gather_sum(idx, tbl, *, vocab):
    n_levels, seq = idx.shape
    n_rows, _, d = tbl.shape
    tile_s = seq // 2

    body = functools.partial(_gather_sum_kernel, tile_s=tile_s)
    out = pl.pallas_call(
        body,
        out_shape=jax.ShapeDtypeStruct((seq, 1, d), jnp.float32),
        grid_spec=pltpu.PrefetchScalarGridSpec(
            num_scalar_prefetch=1,
            grid=(2, n_levels),
            in_specs=[
                pl.BlockSpec((vocab, 1, d), lambda s, l, ids: (l, 0, 0)),
            ],
            out_specs=pl.BlockSpec((tile_s, 1, d), lambda s, l, ids: (s, 0, 0)),
            scratch_shapes=[pltpu.VMEM((tile_s, 1, d), jnp.float32)],
        ),
        compiler_params=pltpu.CompilerParams(
            dimension_semantics=("parallel", "arbitrary"),
            vmem_limit_bytes=64 * 2**20),
    )(idx, tbl)
    return out.reshape(seq, d)


def kernel(xi, tables):
    xi = jnp.asarray(xi)
    n_levels = xi.shape[-1] - 1
    idx = jnp.transpose(xi[:, :n_levels]).astype(jnp.int32)
    n_tbl, n_tok, d = tables.shape
    tbl = tables.reshape(n_tbl * n_tok, 1, d)
    return _embed_gather_sum(idx, tbl, vocab=n_tok)


# speedup vs baseline: 4.0089x; 4.0089x over previous
import functools

import jax
import jax.numpy as jnp
from jax.experimental import pallas as pl
from jax.experimental.pallas import tpu as pltpu


def _level_stream_kernel(ids_ref, tbl_ref, o_ref, *, vocab):
    l = pl.program_id(1)
    ids = ids_ref[l, :]
    tok = jax.lax.broadcasted_iota(jnp.int32, (1, vocab), 1)
    onehot = (ids[:, None] == tok).astype(jnp.bfloat16)
    part = jnp.dot(onehot, tbl_ref[...].astype(jnp.bfloat16),
                   preferred_element_type=jnp.float32)

    @pl.when(l == 0)
    def _():
        o_ref[...] = part

    @pl.when(l > 0)
    def _():
        o_ref[...] += part


@functools.partial(jax.jit, static_argnames=("vocab",))
def _embed_sum(idx, tbl, *, vocab):
    n_levels, seq = idx.shape
    _, d = tbl.shape
    tile_s = seq // 2

    body = functools.partial(_level_stream_kernel, vocab=vocab)
    return pl.pallas_call(
        body,
        out_shape=jax.ShapeDtypeStruct((seq, d), jnp.float32),
        grid=(2, n_levels),
        in_specs=[
            pl.BlockSpec((n_levels, tile_s), lambda s, l: (0, s)),
            pl.BlockSpec((vocab, d), lambda s, l: (l, 0)),
        ],
        out_specs=pl.BlockSpec((tile_s, d), lambda s, l: (s, 0)),
        compiler_params=pltpu.CompilerParams(
            dimension_semantics=("parallel", "arbitrary"),
            vmem_limit_bytes=64 * 2**20),
    )(idx, tbl)


def kernel(xi, tables):
    xi = jnp.asarray(xi)
    n_levels = xi.shape[-1] - 1
    idx = jnp.transpose(xi[:, :n_levels]).astype(jnp.int32)
    n_tbl, n_tok, d = tables.shape
    tbl = tables.reshape(n_tbl * n_tok, d)
    return _embed_sum(idx, tbl, vocab=n_tok)
